# baseline (speedup 1.0000x reference)
import jax
import jax.numpy as jnp
from jax.experimental import pallas as pl
from jax.experimental.pallas import tpu as pltpu

B, N, M, D = 4, 4096, 4096, 3
BN = 512
NB = N // BN


def _chamfer_kernel(a_ref, bt_ref, out_ref, colmin_ref):
    b_idx = pl.program_id(0)
    n_idx = pl.program_id(1)

    a = a_ref[0]
    bt = bt_ref[0]

    a2 = jnp.sum(a * a, axis=1, keepdims=True)
    b2 = jnp.sum(bt * bt, axis=0, keepdims=True)

    ab = jnp.float32(0.0)
    abf = a.astype(jnp.bfloat16).astype(jnp.float32)
    bbf = bt.astype(jnp.bfloat16).astype(jnp.float32)
    ab = (abf[:, 0:1] * bbf[0:1, :]
          + abf[:, 1:2] * bbf[1:2, :]
          + abf[:, 2:3] * bbf[2:3, :])

    d = jnp.maximum(a2 + b2 - 2.0 * ab, 0.0)

    row_min = jnp.min(d, axis=1)
    col_min = jnp.min(d, axis=0, keepdims=True)

    @pl.when(n_idx == 0)
    def _():
        colmin_ref[...] = col_min

    @pl.when(n_idx != 0)
    def _():
        colmin_ref[...] = jnp.minimum(colmin_ref[...], col_min)

    partial = jnp.sum(jnp.sqrt(row_min)).reshape(1, 1)

    @pl.when((b_idx == 0) & (n_idx == 0))
    def _():
        out_ref[...] = jnp.zeros((1, 1), jnp.float32)

    out_ref[...] += partial

    @pl.when(n_idx == NB - 1)
    def _():
        out_ref[...] += jnp.sum(jnp.sqrt(colmin_ref[...])).reshape(1, 1)


def kernel(array1, array2):
    bt = jnp.transpose(array2, (0, 2, 1))

    total = pl.pallas_call(
        _chamfer_kernel,
        grid=(B, NB),
        in_specs=[
            pl.BlockSpec((1, BN, D), lambda b, n: (b, n, 0)),
            pl.BlockSpec((1, D, M), lambda b, n: (b, 0, 0)),
        ],
        out_specs=pl.BlockSpec((1, 1), lambda b, n: (0, 0)),
        out_shape=jax.ShapeDtypeStruct((1, 1), jnp.float32),
        scratch_shapes=[pltpu.VMEM((1, M), jnp.float32)],
    )(array1, bt)

    return total[0, 0] / (2.0 * B * N)


# speedup vs baseline: 2.1403x; 2.1403x over previous
import jax
import jax.numpy as jnp
from jax.experimental import pallas as pl
from jax.experimental.pallas import tpu as pltpu

B, N, M, D = 4, 4096, 4096, 3
BN = 512
NB = N // BN


def _chamfer_kernel(a_ref, bt_ref, out_ref, colmin_ref):
    b_idx = pl.program_id(0)
    n_idx = pl.program_id(1)

    a = a_ref[0]
    bt = bt_ref[0]

    a2 = jnp.sum(a * a, axis=1, keepdims=True)
    b2 = jnp.sum(bt * bt, axis=0, keepdims=True)

    am2 = (-2.0 * a).astype(jnp.bfloat16)
    bbf = bt.astype(jnp.bfloat16)

    g = jax.lax.dot_general(
        am2, bbf,
        dimension_numbers=(((1,), (0,)), ((), ())),
        preferred_element_type=jnp.float32,
    )

    t1 = g + b2
    t2 = g + a2

    row_min = jnp.min(t1, axis=1, keepdims=True)
    col_min = jnp.min(t2, axis=0, keepdims=True)

    dist1 = jnp.maximum(row_min + a2, 0.0)
    partial = jnp.sum(jnp.sqrt(dist1)).reshape(1, 1)

    @pl.when(n_idx == 0)
    def _():
        colmin_ref[...] = col_min

    @pl.when(n_idx != 0)
    def _():
        colmin_ref[...] = jnp.minimum(colmin_ref[...], col_min)

    @pl.when((b_idx == 0) & (n_idx == 0))
    def _():
        out_ref[...] = jnp.zeros((1, 1), jnp.float32)

    out_ref[...] += partial

    @pl.when(n_idx == NB - 1)
    def _():
        dist2 = jnp.maximum(colmin_ref[...] + b2, 0.0)
        out_ref[...] += jnp.sum(jnp.sqrt(dist2)).reshape(1, 1)


def kernel(array1, array2):
    bt = jnp.transpose(array2, (0, 2, 1))

    total = pl.pallas_call(
        _chamfer_kernel,
        grid=(B, NB),
        in_specs=[
            pl.BlockSpec((1, BN, D), lambda b, n: (b, n, 0)),
            pl.BlockSpec((1, D, M), lambda b, n: (b, 0, 0)),
        ],
        out_specs=pl.BlockSpec((1, 1), lambda b, n: (0, 0)),
        out_shape=jax.ShapeDtypeStruct((1, 1), jnp.float32),
        scratch_shapes=[pltpu.VMEM((1, M), jnp.float32)],
    )(array1, bt)

    return total[0, 0] / (2.0 * B * N)


# speedup vs baseline: 2.1610x; 1.0097x over previous
import jax
import jax.numpy as jnp
from jax.experimental import pallas as pl
from jax.experimental.pallas import tpu as pltpu

B, N, M, D = 4, 4096, 4096, 3
BN = 1024
NB = N // BN


def _chamfer_kernel(a_ref, bt_ref, out_ref, colmin_ref):
    b_idx = pl.program_id(0)
    n_idx = pl.program_id(1)

    a = a_ref[0]
    bt = bt_ref[0]

    a2 = jnp.sum(a * a, axis=1, keepdims=True)
    b2 = jnp.sum(bt * bt, axis=0, keepdims=True)

    am2 = (-2.0 * a).astype(jnp.bfloat16)
    bbf = bt.astype(jnp.bfloat16)

    g = jax.lax.dot_general(
        am2, bbf,
        dimension_numbers=(((1,), (0,)), ((), ())),
        preferred_element_type=jnp.float32,
    )

    t1 = g + b2
    t2 = g + a2

    row_min = jnp.min(t1, axis=1, keepdims=True)
    col_min = jnp.min(t2, axis=0, keepdims=True)

    dist1 = jnp.maximum(row_min + a2, 0.0)
    partial = jnp.sum(jnp.sqrt(dist1)).reshape(1, 1)

    @pl.when(n_idx == 0)
    def _():
        colmin_ref[...] = col_min

    @pl.when(n_idx != 0)
    def _():
        colmin_ref[...] = jnp.minimum(colmin_ref[...], col_min)

    @pl.when((b_idx == 0) & (n_idx == 0))
    def _():
        out_ref[...] = jnp.zeros((1, 1), jnp.float32)

    out_ref[...] += partial

    @pl.when(n_idx == NB - 1)
    def _():
        dist2 = jnp.maximum(colmin_ref[...] + b2, 0.0)
        out_ref[...] += jnp.sum(jnp.sqrt(dist2)).reshape(1, 1)


def kernel(array1, array2):
    bt = jnp.transpose(array2, (0, 2, 1))

    total = pl.pallas_call(
        _chamfer_kernel,
        grid=(B, NB),
        in_specs=[
            pl.BlockSpec((1, BN, D), lambda b, n: (b, n, 0)),
            pl.BlockSpec((1, D, M), lambda b, n: (b, 0, 0)),
        ],
        out_specs=pl.BlockSpec((1, 1), lambda b, n: (0, 0)),
        out_shape=jax.ShapeDtypeStruct((1, 1), jnp.float32),
        scratch_shapes=[pltpu.VMEM((1, M), jnp.float32)],
    )(array1, bt)

    return total[0, 0] / (2.0 * B * N)


# speedup vs baseline: 2.5843x; 1.1959x over previous
import jax
import jax.numpy as jnp
from jax.experimental import pallas as pl
from jax.experimental.pallas import tpu as pltpu

B, N, M, D = 4, 4096, 4096, 3
BN = 1024
NB = N // BN


def _chamfer_kernel(a_ref, bt_ref, out_ref, colmin_ref):
    b_idx = pl.program_id(0)
    n_idx = pl.program_id(1)

    a = a_ref[0]
    bt = bt_ref[0]

    a2 = jnp.sum(a * a, axis=1, keepdims=True)
    b2 = jnp.sum(bt * bt, axis=0, keepdims=True)

    a2h = a2.astype(jnp.bfloat16)
    a2l = (a2 - a2h.astype(jnp.float32)).astype(jnp.bfloat16)
    b2h = b2.astype(jnp.bfloat16)
    b2l = (b2 - b2h.astype(jnp.float32)).astype(jnp.bfloat16)

    ones_a = jnp.ones((BN, 1), jnp.bfloat16)
    ones_b = jnp.ones((2, M), jnp.bfloat16)

    a_ext = jnp.concatenate(
        [(-2.0 * a).astype(jnp.bfloat16), ones_a, ones_a, a2h, a2l], axis=1
    )
    b_ext = jnp.concatenate(
        [bt.astype(jnp.bfloat16), b2h, b2l, ones_b], axis=0
    )

    d = jax.lax.dot_general(
        a_ext, b_ext,
        dimension_numbers=(((1,), (0,)), ((), ())),
        preferred_element_type=jnp.float32,
    )

    row_min = jnp.min(d, axis=1, keepdims=True)
    col_min = jnp.min(d, axis=0, keepdims=True)

    dist1 = jnp.maximum(row_min, 0.0)
    partial = jnp.sum(jnp.sqrt(dist1)).reshape(1, 1)

    @pl.when(n_idx == 0)
    def _():
        colmin_ref[...] = col_min

    @pl.when(n_idx != 0)
    def _():
        colmin_ref[...] = jnp.minimum(colmin_ref[...], col_min)

    @pl.when((b_idx == 0) & (n_idx == 0))
    def _():
        out_ref[...] = jnp.zeros((1, 1), jnp.float32)

    out_ref[...] += partial

    @pl.when(n_idx == NB - 1)
    def _():
        dist2 = jnp.maximum(colmin_ref[...], 0.0)
        out_ref[...] += jnp.sum(jnp.sqrt(dist2)).reshape(1, 1)


def kernel(array1, array2):
    bt = jnp.transpose(array2, (0, 2, 1))

    total = pl.pallas_call(
        _chamfer_kernel,
        grid=(B, NB),
        in_specs=[
            pl.BlockSpec((1, BN, D), lambda b, n: (b, n, 0)),
            pl.BlockSpec((1, D, M), lambda b, n: (b, 0, 0)),
        ],
        out_specs=pl.BlockSpec((1, 1), lambda b, n: (0, 0)),
        out_shape=jax.ShapeDtypeStruct((1, 1), jnp.float32),
        scratch_shapes=[pltpu.VMEM((1, M), jnp.float32)],
    )(array1, bt)

    return total[0, 0] / (2.0 * B * N)
